# Initial kernel scaffold; baseline (speedup 1.0000x reference)
import jax
import jax.numpy as jnp
from jax.experimental import pallas as pl


def kernel(x, tables):
    raise NotImplementedError("write your pallas kernel here")


# speedup vs baseline: 23.8839x; 23.8839x over previous
import functools

import jax
import jax.numpy as jnp
from jax import lax
from jax.experimental import pallas as pl
from jax.experimental.pallas import tpu as pltpu
from jax.experimental.pallas import tpu_sc as plsc

N_NODES = 100000
N_FEATS = 9
VOCAB = 119
EMB = 128

NC = 2
NS = 16
NW = NC * NS

C = 40
NCHUNKS = N_NODES // C
CHUNKS_PER_W = -(-NCHUNKS // NW)
LANES = 16
VPR = EMB // LANES

_mesh = plsc.VectorSubcoreMesh(core_axis_name="c", subcore_axis_name="s")


@functools.partial(
    pl.kernel,
    mesh=_mesh,
    out_type=jax.ShapeDtypeStruct((N_NODES, EMB), jnp.float32),
    scratch_types=[
        pltpu.VMEM((N_FEATS, C), jnp.int32),
        pltpu.VMEM((N_FEATS, C, EMB), jnp.float32),
        pltpu.VMEM((C, EMB), jnp.float32),
        pltpu.SemaphoreType.DMA,
    ],
)
def _encode(xt_hbm, tbl_hbm, out_hbm, idx_v, rows_v, out_v, sem):
    wid = lax.axis_index("s") * NC + lax.axis_index("c")

    def chunk_body(i):
        ch = i * NW + wid

        @pl.when(ch < NCHUNKS)
        def _():
            base = ch * C
            pltpu.sync_copy(xt_hbm.at[ch], idx_v)
            copies = [
                pltpu.async_copy(tbl_hbm.at[idx_v.at[f]], rows_v.at[f], sem)
                for f in range(N_FEATS)
            ]
            for cp in copies:
                cp.wait()

            def node_body(n):
                for v in range(VPR):
                    sl = pl.ds(v * LANES, LANES)
                    acc = rows_v[0, n, sl]
                    for f in range(1, N_FEATS):
                        acc = acc + rows_v[f, n, sl]
                    out_v[n, sl] = acc

            pl.loop(0, C)(node_body)
            pltpu.sync_copy(out_v, out_hbm.at[pl.ds(base, C)])

    pl.loop(0, CHUNKS_PER_W)(chunk_body)


def kernel(x, tables):
    offs = (jnp.arange(N_FEATS, dtype=jnp.int32) * VOCAB)[:, None]
    xt = x.astype(jnp.int32).T + offs
    xt3 = xt.reshape(N_FEATS, NCHUNKS, C).transpose(1, 0, 2)
    tbl = tables.reshape(N_FEATS * VOCAB, EMB)
    return _encode(xt3, tbl)


# speedup vs baseline: 29.9777x; 1.2551x over previous
import functools

import jax
import jax.numpy as jnp
from jax import lax
from jax.experimental import pallas as pl
from jax.experimental.pallas import tpu as pltpu
from jax.experimental.pallas import tpu_sc as plsc

N_NODES = 100000
N_FEATS = 9
VOCAB = 119
EMB = 128

NC = 2
NS = 16
NW = NC * NS

C = 80
NCHUNKS = N_NODES // C
CHUNKS_PER_W = -(-NCHUNKS // NW)
NITER = CHUNKS_PER_W + (CHUNKS_PER_W % 2)
LANES = 16
VPR = EMB // LANES

_mesh = plsc.VectorSubcoreMesh(core_axis_name="c", subcore_axis_name="s")


@functools.partial(
    pl.kernel,
    mesh=_mesh,
    out_type=jax.ShapeDtypeStruct((N_NODES, EMB), jnp.float32),
    scratch_types=[
        pltpu.VMEM((2, N_FEATS, C), jnp.int32),
        pltpu.VMEM((2, C, EMB), jnp.float32),
        pltpu.SemaphoreType.DMA,
        pltpu.SemaphoreType.DMA,
        pltpu.SemaphoreType.DMA,
        pltpu.SemaphoreType.DMA,
    ],
)
def _encode(xt_hbm, tbl_hbm, out_hbm, idx_v, acc_v, g0, g1, o0, o1):
    wid = lax.axis_index("s") * NC + lax.axis_index("c")
    gsems = (g0, g1)
    osems = (o0, o1)

    zeros = jnp.zeros((LANES,), jnp.float32)

    def start(i, b):
        ch = i * NW + wid

        @pl.when(ch < NCHUNKS)
        def _():
            def zero_node(n):
                for v in range(VPR):
                    acc_v[b, n, pl.ds(v * LANES, LANES)] = zeros

            pl.loop(0, C)(zero_node)
            pltpu.sync_copy(xt_hbm.at[ch], idx_v.at[b])
            for f in range(N_FEATS):
                pltpu.async_copy(
                    tbl_hbm.at[idx_v.at[b, f]], acc_v.at[b], gsems[b], add=True
                )

    def finish(i, b):
        ch = i * NW + wid

        @pl.when((ch >= 0) & (ch < NCHUNKS))
        def _():
            for f in range(N_FEATS):
                pltpu.make_async_copy(
                    tbl_hbm.at[idx_v.at[b, f]], acc_v.at[b], gsems[b]
                ).wait()
            pltpu.async_copy(acc_v.at[b], out_hbm.at[pl.ds(ch * C, C)], osems[b])

    def drain_out(i, b):
        ch = i * NW + wid

        @pl.when((ch >= 0) & (ch < NCHUNKS))
        def _():
            pltpu.make_async_copy(
                acc_v.at[b], out_hbm.at[pl.ds(ch * C, C)], osems[b]
            ).wait()

    def body(i2):
        for b in range(2):
            i = i2 + b
            drain_out(i - 2, b)
            start(i, b)
            finish(i - 1, 1 - b)

    pl.loop(0, NITER, step=2)(body)
    finish(NITER - 1, (NITER - 1) % 2)
    drain_out(NITER - 2, (NITER - 2) % 2)
    drain_out(NITER - 1, (NITER - 1) % 2)


def kernel(x, tables):
    offs = (jnp.arange(N_FEATS, dtype=jnp.int32) * VOCAB)[:, None]
    xt = x.astype(jnp.int32).T + offs
    xt3 = xt.reshape(N_FEATS, NCHUNKS, C).transpose(1, 0, 2)
    tbl = tables.reshape(N_FEATS * VOCAB, EMB)
    return _encode(xt3, tbl)
